# baseline (speedup 1.0000x reference)
import functools
import jax
import jax.numpy as jnp
from jax import lax
from jax.experimental import pallas as pl
from jax.experimental.pallas import tpu as pltpu
from jax.experimental.pallas import tpu_sc as plsc

C_IN = 16
C_OUT = 32
KSIZE = 2
NUM_CELLS = KSIZE * 8
RADIUS = 0.2
_R2 = RADIUS * RADIUS
_W2 = (RADIUS / KSIZE) * (RADIUS / KSIZE)

_B = 2
_N = 2048
_NC = 2
_NS = 16
_NW = _NC * _NS
_QPW = (_B * _N) // _NW
_WPB = _NW // _B
_GCOLS = NUM_CELLS * C_IN
_PAIR_CAP = _N + 16


def _sc_body(pts_hbm, attrs_hbm, out_hbm,
             pts_v, attrs_v, pairs_v, acc_v, cnt_v, outb_v, sem):
    cid = lax.axis_index("c")
    sid = lax.axis_index("s")
    wid = sid * _NC + cid
    batch = wid // _WPB
    lq0 = (wid % _WPB) * _QPW

    pltpu.sync_copy(pts_hbm.at[batch], pts_v)
    pltpu.sync_copy(attrs_hbm.at[pl.ds(batch * (_N * C_IN), _N * C_IN)],
                    attrs_v)

    zeros16 = jnp.zeros((16,), jnp.float32)
    ones16 = jnp.ones((16,), jnp.float32)
    lane = lax.iota(jnp.int32, 16)
    zero_i = jnp.zeros((16,), jnp.int32)
    one_i = jnp.full((16,), 1, jnp.int32)
    two_i = jnp.full((16,), 2, jnp.int32)

    pad16 = jnp.full((16,), 4095, jnp.int32)

    def per_query(qi, carry):
        i = lq0 + qi
        iv = jnp.full((16,), 1, jnp.int32) * i
        qx = plsc.load_gather(pts_v, [zero_i, iv])
        qy = plsc.load_gather(pts_v, [one_i, iv])
        qz = plsc.load_gather(pts_v, [two_i, iv])
        for c in range(NUM_CELLS + 1):
            acc_v[pl.ds(c * 16, 16)] = zeros16
            cnt_v[pl.ds(c * 16, 16)] = zeros16

        def chunk(ck, npairs):
            j0 = ck * 16
            xv = pts_v[0, pl.ds(j0, 16)]
            yv = pts_v[1, pl.ds(j0, 16)]
            zv = pts_v[2, pl.ds(j0, 16)]
            dx = qx - xv
            dy = qy - yv
            dz = qz - zv
            d2e = dx * dx + dy * dy + dz * dz + jnp.float32(1e-12)
            inrad = d2e < jnp.float32(_R2)
            plsc.store_compressed(pairs_v.at[pl.ds(npairs, 16)], lane + j0,
                                  mask=inrad)
            cntv = plsc.all_reduce_population_count(inrad)
            return npairs + cntv[0]

        np_total = lax.fori_loop(0, _N // 16, chunk, jnp.int32(0))
        pairs_v[pl.ds(np_total, 16)] = pad16
        ngroups = (np_total + 15) // 16

        def pair_group(g, _c):
            pv = pairs_v[pl.ds(g * 16, 16)]
            jc = pv & jnp.int32(_N - 1)
            xj = plsc.load_gather(pts_v, [zero_i, jc])
            yj = plsc.load_gather(pts_v, [one_i, jc])
            zj = plsc.load_gather(pts_v, [two_i, jc])
            dx = qx - xj
            dy = qy - yj
            dz = qz - zj
            d2e = dx * dx + dy * dy + dz * dz + jnp.float32(1e-12)
            cellv = (jnp.where(d2e >= jnp.float32(_W2), 8, 0)
                     + jnp.where(dx >= 0, 4, 0)
                     + jnp.where(dy >= 0, 2, 0)
                     + jnp.where(dz >= 0, 1, 0))
            cellv = jnp.where(pv >= jnp.int32(_N), NUM_CELLS, cellv)
            c16v = cellv << 4
            j16v = jc << 4
            for l in range(16):
                plsc.addupdate(acc_v.at[pl.ds(c16v[l], 16)],
                               attrs_v[pl.ds(j16v[l], 16)])
                plsc.addupdate(cnt_v.at[pl.ds(c16v[l], 16)], ones16)
            return _c

        lax.fori_loop(0, ngroups, pair_group, jnp.int32(0))

        row0 = qi * _GCOLS
        for c in range(NUM_CELLS):
            denom = jnp.maximum(cnt_v[pl.ds(c * 16, 16)], jnp.float32(1.0))
            outb_v[pl.ds(row0 + c * 16, 16)] = acc_v[pl.ds(c * 16, 16)] / denom
        return carry

    lax.fori_loop(0, _QPW, per_query, 0)
    pltpu.sync_copy(outb_v, out_hbm.at[pl.ds(wid * _QPW * _GCOLS,
                                             _QPW * _GCOLS)])


def _make_sc():
    mesh = plsc.VectorSubcoreMesh(core_axis_name="c", subcore_axis_name="s")
    return functools.partial(
        pl.kernel,
        out_type=jax.ShapeDtypeStruct((_B * _N * _GCOLS,), jnp.float32),
        mesh=mesh,
        compiler_params=pltpu.CompilerParams(needs_layout_passes=False),
        scratch_types=[
            pltpu.VMEM((3, _N), jnp.float32),
            pltpu.VMEM((_N * C_IN,), jnp.float32),
            pltpu.VMEM((_PAIR_CAP,), jnp.int32),
            pltpu.VMEM(((NUM_CELLS + 1) * 16,), jnp.float32),
            pltpu.VMEM(((NUM_CELLS + 1) * 16,), jnp.float32),
            pltpu.VMEM((_QPW * _GCOLS,), jnp.float32),
            pltpu.SemaphoreType.DMA,
        ],
    )(_sc_body)


def _conv_body(g_ref, w_ref, b_ref, out_ref):
    out_ref[...] = jax.lax.dot_general(
        g_ref[...], w_ref[...], (((1,), (0,)), ((), ())),
        preferred_element_type=jnp.float32,
        precision=jax.lax.Precision.HIGHEST) + b_ref[0]


def kernel(points_tensor, batch_atributes, W, b):
    B, N, _ = points_tensor.shape
    pts_t = jnp.transpose(points_tensor, (0, 2, 1))
    g = _make_sc()(pts_t, batch_atributes.reshape(B * N * C_IN)
                   ).reshape(B * N, _GCOLS)

    w_flat = jnp.transpose(W, (2, 1, 0)).reshape(_GCOLS, C_OUT)
    b2 = b.reshape(1, C_OUT)
    out = pl.pallas_call(
        _conv_body,
        grid=(B * N // 512,),
        in_specs=[
            pl.BlockSpec((512, _GCOLS), lambda i: (i, 0)),
            pl.BlockSpec((_GCOLS, C_OUT), lambda i: (0, 0)),
            pl.BlockSpec((1, C_OUT), lambda i: (0, 0)),
        ],
        out_specs=pl.BlockSpec((512, C_OUT), lambda i: (i, 0)),
        out_shape=jax.ShapeDtypeStruct((B * N, C_OUT), jnp.float32),
    )(g, w_flat, b2)
    return out.reshape(B, N, C_OUT)


# speedup vs baseline: 1.1679x; 1.1679x over previous
import functools
import jax
import jax.numpy as jnp
from jax import lax
from jax.experimental import pallas as pl
from jax.experimental.pallas import tpu as pltpu
from jax.experimental.pallas import tpu_sc as plsc

C_IN = 16
C_OUT = 32
KSIZE = 2
NUM_CELLS = KSIZE * 8
RADIUS = 0.2
_R2 = RADIUS * RADIUS
_W2 = (RADIUS / KSIZE) * (RADIUS / KSIZE)

_B = 2
_N = 2048
_NC = 2
_NS = 16
_NW = _NC * _NS
_QPW = (_B * _N) // _NW
_WPB = _NW // _B
_GCOLS = NUM_CELLS * C_IN
_PAIR_CAP = _N + 16


def _sc_body(pts_hbm, attrs_hbm, out_hbm,
             pts_v, attrs_v, raw_v, cnts_v, pairs_v, acc_v, cnt_v, outb_v,
             sem):
    cid = lax.axis_index("c")
    sid = lax.axis_index("s")
    wid = sid * _NC + cid
    batch = wid // _WPB
    lq0 = (wid % _WPB) * _QPW

    pltpu.sync_copy(pts_hbm.at[batch], pts_v)
    pltpu.sync_copy(attrs_hbm.at[pl.ds(batch * (_N * C_IN), _N * C_IN)],
                    attrs_v)

    zeros16 = jnp.zeros((16,), jnp.float32)
    ones16 = jnp.ones((16,), jnp.float32)
    lane = lax.iota(jnp.int32, 16)
    lane0 = lane == 0
    zero_i = jnp.zeros((16,), jnp.int32)
    one_i = jnp.full((16,), 1, jnp.int32)
    two_i = jnp.full((16,), 2, jnp.int32)

    pad16 = jnp.full((16,), 4095, jnp.int32)

    def per_query(qi, carry):
        i = lq0 + qi
        iv = jnp.full((16,), 1, jnp.int32) * i
        qx = plsc.load_gather(pts_v, [zero_i, iv])
        qy = plsc.load_gather(pts_v, [one_i, iv])
        qz = plsc.load_gather(pts_v, [two_i, iv])
        for c in range(NUM_CELLS + 1):
            acc_v[pl.ds(c * 16, 16)] = zeros16
            cnt_v[pl.ds(c * 16, 16)] = zeros16

        @plsc.parallel_loop(0, _N // 16, 1, unroll=4)
        def _scan(ck):
            j0 = ck * 16
            xv = pts_v[0, pl.ds(j0, 16)]
            yv = pts_v[1, pl.ds(j0, 16)]
            zv = pts_v[2, pl.ds(j0, 16)]
            dx = qx - xv
            dy = qy - yv
            dz = qz - zv
            d2e = dx * dx + dy * dy + dz * dz + jnp.float32(1e-12)
            inrad = d2e < jnp.float32(_R2)
            plsc.store_compressed(raw_v.at[pl.ds(j0, 16)], lane + j0,
                                  mask=inrad)
            cntv = plsc.all_reduce_population_count(inrad)
            plsc.store_compressed(cnts_v.at[pl.ds(ck, 16)], cntv, mask=lane0)

        def stitch(cg, off0):
            cvec = cnts_v[pl.ds(cg * 16, 16)]
            incl = plsc.cumsum(cvec)
            excl = off0 + incl - cvec
            for l in range(16):
                pv = raw_v[pl.ds((cg * 16 + l) * 16, 16)]
                m = lane < cvec[l]
                plsc.store_compressed(pairs_v.at[pl.ds(excl[l], 16)], pv,
                                      mask=m)
            return off0 + incl[15]

        np_total = lax.fori_loop(0, _N // 256, stitch, jnp.int32(0))
        pairs_v[pl.ds(np_total, 16)] = pad16
        ngroups = (np_total + 15) // 16

        def pair_group(g, _c):
            pv = pairs_v[pl.ds(g * 16, 16)]
            jc = pv & jnp.int32(_N - 1)
            xj = plsc.load_gather(pts_v, [zero_i, jc])
            yj = plsc.load_gather(pts_v, [one_i, jc])
            zj = plsc.load_gather(pts_v, [two_i, jc])
            dx = qx - xj
            dy = qy - yj
            dz = qz - zj
            d2e = dx * dx + dy * dy + dz * dz + jnp.float32(1e-12)
            cellv = (jnp.where(d2e >= jnp.float32(_W2), 8, 0)
                     + jnp.where(dx >= 0, 4, 0)
                     + jnp.where(dy >= 0, 2, 0)
                     + jnp.where(dz >= 0, 1, 0))
            cellv = jnp.where(pv >= jnp.int32(_N), NUM_CELLS, cellv)
            c16v = cellv << 4
            j16v = jc << 4
            for l in range(16):
                plsc.addupdate(acc_v.at[pl.ds(c16v[l], 16)],
                               attrs_v[pl.ds(j16v[l], 16)])
                plsc.addupdate(cnt_v.at[pl.ds(c16v[l], 16)], ones16)
            return _c

        lax.fori_loop(0, ngroups, pair_group, jnp.int32(0))

        row0 = qi * _GCOLS
        for c in range(NUM_CELLS):
            denom = jnp.maximum(cnt_v[pl.ds(c * 16, 16)], jnp.float32(1.0))
            outb_v[pl.ds(row0 + c * 16, 16)] = acc_v[pl.ds(c * 16, 16)] / denom
        return carry

    lax.fori_loop(0, _QPW, per_query, 0)
    pltpu.sync_copy(outb_v, out_hbm.at[pl.ds(wid * _QPW * _GCOLS,
                                             _QPW * _GCOLS)])


def _make_sc():
    mesh = plsc.VectorSubcoreMesh(core_axis_name="c", subcore_axis_name="s")
    return functools.partial(
        pl.kernel,
        out_type=jax.ShapeDtypeStruct((_B * _N * _GCOLS,), jnp.float32),
        mesh=mesh,
        compiler_params=pltpu.CompilerParams(needs_layout_passes=False),
        scratch_types=[
            pltpu.VMEM((3, _N), jnp.float32),
            pltpu.VMEM((_N * C_IN,), jnp.float32),
            pltpu.VMEM((_N,), jnp.int32),
            pltpu.VMEM((_N // 16 + 16,), jnp.int32),
            pltpu.VMEM((_PAIR_CAP,), jnp.int32),
            pltpu.VMEM(((NUM_CELLS + 1) * 16,), jnp.float32),
            pltpu.VMEM(((NUM_CELLS + 1) * 16,), jnp.float32),
            pltpu.VMEM((_QPW * _GCOLS,), jnp.float32),
            pltpu.SemaphoreType.DMA,
        ],
    )(_sc_body)


def _conv_body(g_ref, w_ref, b_ref, out_ref):
    out_ref[...] = jax.lax.dot_general(
        g_ref[...], w_ref[...], (((1,), (0,)), ((), ())),
        preferred_element_type=jnp.float32,
        precision=jax.lax.Precision.HIGHEST) + b_ref[0]


def kernel(points_tensor, batch_atributes, W, b):
    B, N, _ = points_tensor.shape
    pts_t = jnp.transpose(points_tensor, (0, 2, 1))
    g = _make_sc()(pts_t, batch_atributes.reshape(B * N * C_IN)
                   ).reshape(B * N, _GCOLS)

    w_flat = jnp.transpose(W, (2, 1, 0)).reshape(_GCOLS, C_OUT)
    b2 = b.reshape(1, C_OUT)
    out = pl.pallas_call(
        _conv_body,
        grid=(B * N // 512,),
        in_specs=[
            pl.BlockSpec((512, _GCOLS), lambda i: (i, 0)),
            pl.BlockSpec((_GCOLS, C_OUT), lambda i: (0, 0)),
            pl.BlockSpec((1, C_OUT), lambda i: (0, 0)),
        ],
        out_specs=pl.BlockSpec((512, C_OUT), lambda i: (i, 0)),
        out_shape=jax.ShapeDtypeStruct((B * N, C_OUT), jnp.float32),
    )(g, w_flat, b2)
    return out.reshape(B, N, C_OUT)
